# baseline (speedup 1.0000x reference)
import functools

import jax
import jax.numpy as jnp
from jax import lax
from jax.experimental import pallas as pl
from jax.experimental.pallas import tpu as pltpu
from jax.experimental.pallas import tpu_sc as plsc

D = 8
KD = 8
FEAT_DIM = 1024
HIDDEN = 512
N_TOKENS = 4096
N_GLOBAL = D * KD
ZPAD = 128
B = 2048
NB = (N_TOKENS + D * B) // B
NSLOTS = NB * B
ROWS = 32
NW = 32
TPW = N_TOKENS // NW


def _route_kernel(dom_ref, slot_ref, meta_ref):
    dom = dom_ref[...]
    r128 = lax.broadcasted_iota(jnp.int32, (128, 128), 0)
    c128 = lax.broadcasted_iota(jnp.int32, (128, 128), 1)
    upper = jnp.where(r128 < c128, 1.0, 0.0)
    r32 = lax.broadcasted_iota(jnp.int32, (ROWS, ROWS), 0)
    c32 = lax.broadcasted_iota(jnp.int32, (ROWS, ROWS), 1)
    lower = jnp.where(c32 < r32, 1.0, 0.0)
    slot = jnp.zeros(dom.shape, jnp.int32)
    cum = jnp.int32(0)
    cums = []
    for d in range(D):
        oh = jnp.where(dom == d, 1.0, 0.0)
        rowpref = jnp.dot(oh, upper, preferred_element_type=jnp.float32)
        rowtot = jnp.sum(oh, axis=1, keepdims=True)
        colpref = jnp.dot(lower, rowtot, preferred_element_type=jnp.float32)
        rank = (rowpref + colpref).astype(jnp.int32)
        cnt = jnp.sum(rowtot).astype(jnp.int32)
        padded = ((cnt + B - 1) // B) * B
        slot = slot + jnp.where(dom == d, cum + rank, 0)
        cum = cum + padded
        cums.append(cum)
    slot_ref[...] = slot
    lane8 = lax.broadcasted_iota(jnp.int32, (8, 128), 1)
    row8 = lax.broadcasted_iota(jnp.int32, (8, 128), 0)
    nact = cums[D - 1] // B
    eff = jnp.minimum(lane8, nact - 1)
    bstart = jnp.minimum(lane8 * B, cums[D - 1] - B)
    bd = jnp.zeros((8, 128), jnp.int32)
    for d in range(D):
        bd = bd + jnp.where(bstart >= cums[d], 1, 0)
    bd = jnp.minimum(bd, D - 1)
    active = jnp.where((lane8 * B < cums[D - 1]) & (lane8 < NB), 1, 0)
    meta_ref[...] = jnp.where(
        row8 == 0, bd,
        jnp.where(row8 == 1, active, jnp.where(row8 == 2, eff, 0)))


def _route(dom):
    return pl.pallas_call(
        _route_kernel,
        out_shape=(jax.ShapeDtypeStruct((ROWS, 128), jnp.int32),
                   jax.ShapeDtypeStruct((8, 128), jnp.int32)),
    )(dom)


_FC = 32
_NBUF = 3


def _sc_dispatch_body(feats_hbm, z_hbm, slot4_hbm, slot3_hbm, xs_out, zs_out,
                      idx4, zidx, fb0, fb1, fb2, zbuf,
                      fsem0, fsem1, fsem2, zsem):
    wid = lax.axis_index("s") * 2 + lax.axis_index("c")
    base = wid * TPW
    pltpu.sync_copy(slot4_hbm.at[wid], idx4)
    pltpu.sync_copy(slot3_hbm.at[wid], zidx)
    pltpu.sync_copy(z_hbm.at[pl.ds(base, TPW)], zbuf)
    zcp = pltpu.async_copy(zbuf, zs_out.at[zidx.at[0]], zsem)
    fbufs = (fb0, fb1, fb2)
    sems = (fsem0, fsem1, fsem2)
    copies = [None] * _NBUF
    for c in range(TPW // _FC):
        k = c % _NBUF
        if copies[k] is not None:
            copies[k].wait()
        pltpu.sync_copy(feats_hbm.at[pl.ds(base + c * _FC, _FC)], fbufs[k])
        copies[k] = pltpu.async_copy(fbufs[k], xs_out.at[idx4.at[c]], sems[k])
    for k in range(_NBUF):
        copies[k].wait()
    zcp.wait()


def _sc_combine_body(dzs_hbm, slot2_hbm, out_hbm, idx_v, buf, sem):
    wid = lax.axis_index("s") * 2 + lax.axis_index("c")
    base = wid * TPW
    pltpu.sync_copy(slot2_hbm.at[wid], idx_v)
    pltpu.async_copy(dzs_hbm.at[idx_v], buf, sem).wait()
    pltpu.sync_copy(buf, out_hbm.at[pl.ds(base, TPW)])


def _dispatch(feats, z, slot4, slot3):
    mesh = plsc.VectorSubcoreMesh(core_axis_name="c", subcore_axis_name="s")
    call = pl.kernel(
        _sc_dispatch_body,
        mesh=mesh,
        out_type=(jax.ShapeDtypeStruct((NSLOTS, FEAT_DIM), jnp.float32),
                  jax.ShapeDtypeStruct((NSLOTS, ZPAD), jnp.float32)),
        scratch_types=[
            pltpu.VMEM((TPW // _FC, _FC), jnp.int32),
            pltpu.VMEM((1, TPW), jnp.int32),
            pltpu.VMEM((_FC, FEAT_DIM), jnp.float32),
            pltpu.VMEM((_FC, FEAT_DIM), jnp.float32),
            pltpu.VMEM((_FC, FEAT_DIM), jnp.float32),
            pltpu.VMEM((TPW, ZPAD), jnp.float32),
            pltpu.SemaphoreType.DMA,
            pltpu.SemaphoreType.DMA,
            pltpu.SemaphoreType.DMA,
            pltpu.SemaphoreType.DMA,
        ],
    )
    return call(feats, z, slot4, slot3)


def _combine(dzs, slot2):
    mesh = plsc.VectorSubcoreMesh(core_axis_name="c", subcore_axis_name="s")
    call = pl.kernel(
        _sc_combine_body,
        mesh=mesh,
        out_type=jax.ShapeDtypeStruct((N_TOKENS, ZPAD), jnp.float32),
        scratch_types=[
            pltpu.VMEM((TPW,), jnp.int32),
            pltpu.VMEM((TPW, ZPAD), jnp.float32),
            pltpu.SemaphoreType.DMA,
        ],
    )
    return call(dzs, slot2)


def _conf_features(z, d):
    lane = lax.broadcasted_iota(jnp.int32, z.shape, 1)
    gmask = (lane // KD) == d
    neg_inf = jnp.float32(-jnp.inf)
    zm = jnp.where(gmask, z, neg_inf)
    m = jnp.max(zm, axis=1, keepdims=True)
    e = jnp.where(gmask, jnp.exp(zm - m), 0.0)
    s = jnp.sum(e, axis=1, keepdims=True)
    p = e / s
    p_max = jnp.max(p, axis=1, keepdims=True)
    logp = jnp.log(jnp.maximum(p, 1e-12))
    ent = -jnp.sum(jnp.where(gmask, p * logp, 0.0), axis=1, keepdims=True)
    is_max = jnp.where(gmask & (p == p_max), 1.0, 0.0)
    n_max = jnp.sum(is_max, axis=1, keepdims=True)
    m2 = jnp.max(jnp.where(gmask & (p != p_max), p, neg_inf), axis=1,
                 keepdims=True)
    margin = jnp.where(n_max > 1, 0.0, p_max - m2)
    return p_max, ent, margin


def _mlp_kernel(bd_ref, act_ref, eff_ref, xs_ref, zs_ref, w1_ref, b1_ref,
                w2_ref, b2_ref, alpha_ref, out_ref):
    i = pl.program_id(0)
    d = bd_ref[i]

    @pl.when(act_ref[i] == 1)
    def _():
        z = zs_ref[...]
        p_max, ent, margin = _conf_features(z, d)
        w1 = w1_ref[0]
        acc = jnp.dot(xs_ref[...].astype(jnp.bfloat16),
                      w1[:FEAT_DIM, :].astype(jnp.bfloat16),
                      preferred_element_type=jnp.float32)
        acc += (p_max * w1[FEAT_DIM:FEAT_DIM + 1, :]
                + ent * w1[FEAT_DIM + 1:FEAT_DIM + 2, :]
                + margin * w1[FEAT_DIM + 2:FEAT_DIM + 3, :])
        h = jnp.maximum(acc + b1_ref[0], 0.0).astype(jnp.bfloat16)
        dz = jnp.dot(h, w2_ref[0], preferred_element_type=jnp.float32)
        out_ref[...] = z + (dz + b2_ref[0]) * alpha_ref[0, 0, 0]


def _mlp(bd, act, eff, xs, zs, w1, b1r, w2big, b2big, alphar):
    grid_spec = pltpu.PrefetchScalarGridSpec(
        num_scalar_prefetch=3,
        grid=(NB,),
        in_specs=[
            pl.BlockSpec((B, FEAT_DIM), lambda i, bd, act, eff: (eff[i], 0)),
            pl.BlockSpec((B, ZPAD), lambda i, bd, act, eff: (eff[i], 0)),
            pl.BlockSpec((1, FEAT_DIM + 3, HIDDEN),
                         lambda i, bd, act, eff: (bd[i], 0, 0)),
            pl.BlockSpec((1, 1, HIDDEN), lambda i, bd, act, eff: (bd[i], 0, 0)),
            pl.BlockSpec((1, HIDDEN, ZPAD),
                         lambda i, bd, act, eff: (bd[i], 0, 0)),
            pl.BlockSpec((1, 1, ZPAD), lambda i, bd, act, eff: (bd[i], 0, 0)),
            pl.BlockSpec((1, 1, 1), lambda i, bd, act, eff: (bd[i], 0, 0)),
        ],
        out_specs=pl.BlockSpec((B, ZPAD), lambda i, bd, act, eff: (eff[i], 0)),
    )
    return pl.pallas_call(
        _mlp_kernel,
        grid_spec=grid_spec,
        out_shape=jax.ShapeDtypeStruct((NSLOTS, ZPAD), jnp.float32),
    )(bd, act, eff, xs, zs, w1, b1r, w2big, b2big, alphar)


@jax.jit
def kernel(z_base_global, domain_ids, feats, W1, b1, W2, b2, alphas):
    dom = domain_ids.astype(jnp.int32).reshape(ROWS, 128)
    slot, meta = _route(dom)
    slot_flat = slot.reshape(-1)
    slot4 = slot_flat.reshape(NW, TPW // _FC, _FC)
    slot3 = slot_flat.reshape(NW, 1, TPW)
    slot2 = slot_flat.reshape(NW, TPW)
    bd = meta[0, :NB]
    act = meta[1, :NB]
    eff = meta[2, :NB]

    z128 = jnp.pad(z_base_global, ((0, 0), (0, ZPAD - N_GLOBAL)))
    xs, zs = _dispatch(feats, z128, slot4, slot3)

    b1r = b1.reshape(D, 1, HIDDEN)
    eye = jnp.eye(D, dtype=W2.dtype)
    w2big = (W2[:, :, None, :] * eye[:, None, :, None]).reshape(
        D, HIDDEN, N_GLOBAL)
    w2big = jnp.pad(w2big, ((0, 0), (0, 0), (0, ZPAD - N_GLOBAL)))
    w2big = w2big.astype(jnp.bfloat16)
    b2big = (b2[:, None, :] * eye[:, :, None]).reshape(D, 1, N_GLOBAL)
    b2big = jnp.pad(b2big, ((0, 0), (0, 0), (0, ZPAD - N_GLOBAL)))
    alphar = alphas.reshape(D, 1, 1)

    dzs = _mlp(bd, act, eff, xs, zs, W1, b1r, w2big, b2big, alphar)
    return _combine(dzs, slot2)[:, :N_GLOBAL]


# speedup vs baseline: 1.2040x; 1.2040x over previous
import functools

import jax
import jax.numpy as jnp
from jax import lax
from jax.experimental import pallas as pl
from jax.experimental.pallas import tpu as pltpu
from jax.experimental.pallas import tpu_sc as plsc

D = 8
KD = 8
FEAT_DIM = 1024
HIDDEN = 512
N_TOKENS = 4096
N_GLOBAL = D * KD
ZPAD = 128
B = 1024
NB = (N_TOKENS + D * B) // B
NSLOTS = NB * B
ROWS = 32
NW = 32
TPW = N_TOKENS // NW


def _route_kernel(dom_ref, slot_ref, meta_ref):
    dom = dom_ref[...]
    r128 = lax.broadcasted_iota(jnp.int32, (128, 128), 0)
    c128 = lax.broadcasted_iota(jnp.int32, (128, 128), 1)
    upper = jnp.where(r128 < c128, 1.0, 0.0)
    r32 = lax.broadcasted_iota(jnp.int32, (ROWS, ROWS), 0)
    c32 = lax.broadcasted_iota(jnp.int32, (ROWS, ROWS), 1)
    lower = jnp.where(c32 < r32, 1.0, 0.0)
    slot = jnp.zeros(dom.shape, jnp.int32)
    cum = jnp.int32(0)
    cums = []
    for d in range(D):
        oh = jnp.where(dom == d, 1.0, 0.0)
        rowpref = jnp.dot(oh, upper, preferred_element_type=jnp.float32)
        rowtot = jnp.sum(oh, axis=1, keepdims=True)
        colpref = jnp.dot(lower, rowtot, preferred_element_type=jnp.float32)
        rank = (rowpref + colpref).astype(jnp.int32)
        cnt = jnp.sum(rowtot).astype(jnp.int32)
        padded = ((cnt + B - 1) // B) * B
        slot = slot + jnp.where(dom == d, cum + rank, 0)
        cum = cum + padded
        cums.append(cum)
    slot_ref[...] = slot
    lane8 = lax.broadcasted_iota(jnp.int32, (8, 128), 1)
    row8 = lax.broadcasted_iota(jnp.int32, (8, 128), 0)
    nact = cums[D - 1] // B
    eff = jnp.minimum(lane8, nact - 1)
    bstart = jnp.minimum(lane8 * B, cums[D - 1] - B)
    bd = jnp.zeros((8, 128), jnp.int32)
    for d in range(D):
        bd = bd + jnp.where(bstart >= cums[d], 1, 0)
    bd = jnp.minimum(bd, D - 1)
    active = jnp.where((lane8 * B < cums[D - 1]) & (lane8 < NB), 1, 0)
    meta_ref[...] = jnp.where(
        row8 == 0, bd,
        jnp.where(row8 == 1, active, jnp.where(row8 == 2, eff, 0)))


def _route(dom):
    return pl.pallas_call(
        _route_kernel,
        out_shape=(jax.ShapeDtypeStruct((ROWS, 128), jnp.int32),
                   jax.ShapeDtypeStruct((8, 128), jnp.int32)),
    )(dom)


_FC = 32
_NBUF = 3


def _sc_dispatch_body(feats_hbm, z_hbm, slot4_hbm, slot3_hbm, xs_out, zs_out,
                      idx4, zidx, fb0, fb1, fb2, zbuf,
                      fsem0, fsem1, fsem2, zsem):
    wid = lax.axis_index("s") * 2 + lax.axis_index("c")
    base = wid * TPW
    pltpu.sync_copy(slot4_hbm.at[wid], idx4)
    pltpu.sync_copy(slot3_hbm.at[wid], zidx)
    pltpu.sync_copy(z_hbm.at[pl.ds(base, TPW)], zbuf)
    zcp = pltpu.async_copy(zbuf, zs_out.at[zidx.at[0]], zsem)
    fbufs = (fb0, fb1, fb2)
    sems = (fsem0, fsem1, fsem2)
    copies = [None] * _NBUF
    for c in range(TPW // _FC):
        k = c % _NBUF
        if copies[k] is not None:
            copies[k].wait()
        pltpu.sync_copy(feats_hbm.at[pl.ds(base + c * _FC, _FC)], fbufs[k])
        copies[k] = pltpu.async_copy(fbufs[k], xs_out.at[idx4.at[c]], sems[k])
    for k in range(_NBUF):
        copies[k].wait()
    zcp.wait()


def _sc_combine_body(dzs_hbm, slot2_hbm, out_hbm, idx_v, buf, sem):
    wid = lax.axis_index("s") * 2 + lax.axis_index("c")
    base = wid * TPW
    pltpu.sync_copy(slot2_hbm.at[wid], idx_v)
    pltpu.async_copy(dzs_hbm.at[idx_v], buf, sem).wait()
    pltpu.sync_copy(buf, out_hbm.at[pl.ds(base, TPW)])


def _dispatch(feats, z, slot4, slot3):
    mesh = plsc.VectorSubcoreMesh(core_axis_name="c", subcore_axis_name="s")
    call = pl.kernel(
        _sc_dispatch_body,
        mesh=mesh,
        out_type=(jax.ShapeDtypeStruct((NSLOTS, FEAT_DIM), jnp.float32),
                  jax.ShapeDtypeStruct((NSLOTS, ZPAD), jnp.float32)),
        scratch_types=[
            pltpu.VMEM((TPW // _FC, _FC), jnp.int32),
            pltpu.VMEM((1, TPW), jnp.int32),
            pltpu.VMEM((_FC, FEAT_DIM), jnp.float32),
            pltpu.VMEM((_FC, FEAT_DIM), jnp.float32),
            pltpu.VMEM((_FC, FEAT_DIM), jnp.float32),
            pltpu.VMEM((TPW, ZPAD), jnp.float32),
            pltpu.SemaphoreType.DMA,
            pltpu.SemaphoreType.DMA,
            pltpu.SemaphoreType.DMA,
            pltpu.SemaphoreType.DMA,
        ],
    )
    return call(feats, z, slot4, slot3)


def _combine(dzs, slot2):
    mesh = plsc.VectorSubcoreMesh(core_axis_name="c", subcore_axis_name="s")
    call = pl.kernel(
        _sc_combine_body,
        mesh=mesh,
        out_type=jax.ShapeDtypeStruct((N_TOKENS, ZPAD), jnp.float32),
        scratch_types=[
            pltpu.VMEM((TPW,), jnp.int32),
            pltpu.VMEM((TPW, ZPAD), jnp.float32),
            pltpu.SemaphoreType.DMA,
        ],
    )
    return call(dzs, slot2)


def _conf_features(z, d):
    lane = lax.broadcasted_iota(jnp.int32, z.shape, 1)
    gmask = (lane // KD) == d
    neg_inf = jnp.float32(-jnp.inf)
    zm = jnp.where(gmask, z, neg_inf)
    m = jnp.max(zm, axis=1, keepdims=True)
    e = jnp.where(gmask, jnp.exp(zm - m), 0.0)
    s = jnp.sum(e, axis=1, keepdims=True)
    p = e / s
    p_max = jnp.max(p, axis=1, keepdims=True)
    logp = jnp.log(jnp.maximum(p, 1e-12))
    ent = -jnp.sum(jnp.where(gmask, p * logp, 0.0), axis=1, keepdims=True)
    is_max = jnp.where(gmask & (p == p_max), 1.0, 0.0)
    n_max = jnp.sum(is_max, axis=1, keepdims=True)
    m2 = jnp.max(jnp.where(gmask & (p != p_max), p, neg_inf), axis=1,
                 keepdims=True)
    margin = jnp.where(n_max > 1, 0.0, p_max - m2)
    return p_max, ent, margin


def _mlp_kernel(bd_ref, act_ref, eff_ref, xs_ref, zs_ref, w1_ref, b1_ref,
                w2_ref, b2_ref, alpha_ref, out_ref):
    i = pl.program_id(0)
    d = bd_ref[i]

    @pl.when(act_ref[i] == 1)
    def _():
        z = zs_ref[...]
        p_max, ent, margin = _conf_features(z, d)
        w1 = w1_ref[0]
        acc = jnp.dot(xs_ref[...].astype(jnp.bfloat16),
                      w1[:FEAT_DIM, :].astype(jnp.bfloat16),
                      preferred_element_type=jnp.float32)
        acc += (p_max * w1[FEAT_DIM:FEAT_DIM + 1, :]
                + ent * w1[FEAT_DIM + 1:FEAT_DIM + 2, :]
                + margin * w1[FEAT_DIM + 2:FEAT_DIM + 3, :])
        h = jnp.maximum(acc + b1_ref[0], 0.0).astype(jnp.bfloat16)
        dz = jnp.dot(h, w2_ref[0], preferred_element_type=jnp.float32)
        out_ref[...] = z + (dz + b2_ref[0]) * alpha_ref[0, 0, 0]


def _mlp(bd, act, eff, xs, zs, w1, b1r, w2big, b2big, alphar):
    grid_spec = pltpu.PrefetchScalarGridSpec(
        num_scalar_prefetch=3,
        grid=(NB,),
        in_specs=[
            pl.BlockSpec((B, FEAT_DIM), lambda i, bd, act, eff: (eff[i], 0)),
            pl.BlockSpec((B, ZPAD), lambda i, bd, act, eff: (eff[i], 0)),
            pl.BlockSpec((1, FEAT_DIM + 3, HIDDEN),
                         lambda i, bd, act, eff: (bd[i], 0, 0)),
            pl.BlockSpec((1, 1, HIDDEN), lambda i, bd, act, eff: (bd[i], 0, 0)),
            pl.BlockSpec((1, HIDDEN, ZPAD),
                         lambda i, bd, act, eff: (bd[i], 0, 0)),
            pl.BlockSpec((1, 1, ZPAD), lambda i, bd, act, eff: (bd[i], 0, 0)),
            pl.BlockSpec((1, 1, 1), lambda i, bd, act, eff: (bd[i], 0, 0)),
        ],
        out_specs=pl.BlockSpec((B, ZPAD), lambda i, bd, act, eff: (eff[i], 0)),
    )
    return pl.pallas_call(
        _mlp_kernel,
        grid_spec=grid_spec,
        out_shape=jax.ShapeDtypeStruct((NSLOTS, ZPAD), jnp.float32),
    )(bd, act, eff, xs, zs, w1, b1r, w2big, b2big, alphar)


@jax.jit
def kernel(z_base_global, domain_ids, feats, W1, b1, W2, b2, alphas):
    dom = domain_ids.astype(jnp.int32).reshape(ROWS, 128)
    slot, meta = _route(dom)
    slot_flat = slot.reshape(-1)
    slot4 = slot_flat.reshape(NW, TPW // _FC, _FC)
    slot3 = slot_flat.reshape(NW, 1, TPW)
    slot2 = slot_flat.reshape(NW, TPW)
    bd = meta[0, :NB]
    act = meta[1, :NB]
    eff = meta[2, :NB]

    z128 = jnp.pad(z_base_global, ((0, 0), (0, ZPAD - N_GLOBAL)))
    xs, zs = _dispatch(feats, z128, slot4, slot3)

    b1r = b1.reshape(D, 1, HIDDEN)
    eye = jnp.eye(D, dtype=W2.dtype)
    w2big = (W2[:, :, None, :] * eye[:, None, :, None]).reshape(
        D, HIDDEN, N_GLOBAL)
    w2big = jnp.pad(w2big, ((0, 0), (0, 0), (0, ZPAD - N_GLOBAL)))
    w2big = w2big.astype(jnp.bfloat16)
    b2big = (b2[:, None, :] * eye[:, :, None]).reshape(D, 1, N_GLOBAL)
    b2big = jnp.pad(b2big, ((0, 0), (0, 0), (0, ZPAD - N_GLOBAL)))
    alphar = alphas.reshape(D, 1, 1)

    dzs = _mlp(bd, act, eff, xs, zs, W1, b1r, w2big, b2big, alphar)
    return _combine(dzs, slot2)[:, :N_GLOBAL]
